# Initial kernel scaffold; baseline (speedup 1.0000x reference)
import jax
import jax.numpy as jnp
from jax.experimental import pallas as pl


def kernel(distances, type_i, type_j, w):
    raise NotImplementedError("write your pallas kernel here")


# speedup vs baseline: 1.8278x; 1.8278x over previous
import jax
import jax.numpy as jnp
from jax import lax
from jax.experimental import pallas as pl
from jax.experimental.pallas import tpu as pltpu
from jax.experimental.pallas import tpu_sc as plsc

N_TYPES = 100
NUM_RBF = 16
CUTOFF = 5.0
N_BASIS_SET = 2
NUM_EDGES = 1600000
N_PAIRS = N_TYPES * (N_TYPES + 1) // 2

_GAMMA = float((NUM_RBF / CUTOFF) ** 2)
_CENTERS = [i * (CUTOFF / (NUM_RBF - 1)) for i in range(NUM_RBF)]

_CUT_POLY = [
    1.0, -2.467400550842285, 2.0293474197387695, -0.6675792336463928,
    0.11751490086317062, -0.012679492123425007, 0.0007969553698785603,
]

_C = 800
_NCHUNKS = NUM_EDGES // _C
_L = 16


def _sc_kernel_body(d_hbm, ti_hbm, tj_hbm, w_hbm, out_hbm,
                    w_v, d_v, ti_v, tj_v, out_v):
    info = plsc.get_sparse_core_info()
    nw = info.num_cores * info.num_subcores
    wid = lax.axis_index("s") * info.num_cores + lax.axis_index("c")

    pltpu.sync_copy(w_hbm, w_v)

    lanes = lax.broadcasted_iota(jnp.int32, (_L,), 0)
    my_nchunks = (_NCHUNKS - wid + nw - 1) // nw

    def chunk_body(n, carry):
        chunk = wid + n * nw
        base = chunk * _C
        pltpu.sync_copy(d_hbm.at[pl.ds(base, _C)], d_v)
        pltpu.sync_copy(ti_hbm.at[pl.ds(base, _C)], ti_v)
        pltpu.sync_copy(tj_hbm.at[pl.ds(base, _C)], tj_v)

        def group_body(g, carry2):
            off = g * _L
            d16 = d_v[pl.ds(off, _L)]
            ti16 = ti_v[pl.ds(off, _L)]
            tj16 = tj_v[pl.ds(off, _L)]
            i_ = jnp.minimum(ti16, tj16)
            j_ = jnp.maximum(ti16, tj16)
            k16 = ((2 * N_TYPES - i_ + 1) * i_ >> 1) + (j_ - i_)
            kf = k16 * NUM_RBF
            x = d16 * (1.0 / CUTOFF)
            u = x * x
            p = jnp.full((_L,), _CUT_POLY[-1], jnp.float32)
            for c in reversed(_CUT_POLY[:-1]):
                p = p * u + c
            cut16 = jnp.where(d16 < CUTOFF, p, jnp.zeros((_L,), jnp.float32))
            e16 = (off + lanes) * NUM_RBF
            for r in range(NUM_RBF):
                colw = plsc.load_gather(w_v, [kf + r])
                regc = jnp.minimum(jnp.maximum(colw, 0.0), 1.0)
                t = d16 - _CENTERS[r]
                gr = jnp.exp(t * t * (-_GAMMA))
                plsc.store_scatter(out_v, [e16 + r], gr * cut16 * regc)
            return carry2

        lax.fori_loop(0, _C // _L, group_body, 0)
        pltpu.sync_copy(out_v, out_hbm.at[pl.ds(base * NUM_RBF, _C * NUM_RBF)])
        pltpu.sync_copy(
            out_v,
            out_hbm.at[pl.ds(NUM_EDGES * NUM_RBF + base * NUM_RBF,
                             _C * NUM_RBF)])
        return carry

    lax.fori_loop(0, my_nchunks, chunk_body, 0)


def kernel(distances, type_i, type_j, w):
    mesh = plsc.VectorSubcoreMesh(core_axis_name="c", subcore_axis_name="s")
    f = pl.kernel(
        _sc_kernel_body,
        mesh=mesh,
        compiler_params=pltpu.CompilerParams(needs_layout_passes=False),
        out_type=jax.ShapeDtypeStruct((N_BASIS_SET * NUM_EDGES * NUM_RBF,),
                                      jnp.float32),
        scratch_types=[
            pltpu.VMEM((N_PAIRS * NUM_RBF,), jnp.float32),
            pltpu.VMEM((_C,), jnp.float32),
            pltpu.VMEM((_C,), jnp.int32),
            pltpu.VMEM((_C,), jnp.int32),
            pltpu.VMEM((_C * NUM_RBF,), jnp.float32),
        ],
    )
    out_flat = f(distances, type_i, type_j, w.reshape(-1))
    return out_flat.reshape(N_BASIS_SET, NUM_EDGES, NUM_RBF)
